# Initial kernel scaffold; baseline (speedup 1.0000x reference)
import jax
import jax.numpy as jnp
from jax.experimental import pallas as pl


def kernel(gating_logits):
    raise NotImplementedError("write your pallas kernel here")


# speedup vs baseline: 1.2476x; 1.2476x over previous
import functools

import jax
import jax.numpy as jnp
from jax import lax
from jax.experimental import pallas as pl
from jax.experimental.pallas import tpu as pltpu
from jax.experimental.pallas import tpu_sc as plsc

N_TOKENS = 32768
N_EXPERTS = 64
TOPK = 8
NC, NS, L = 2, 16, 16
NW = NC * NS
TPW = N_TOKENS // NW

_GATHER_DNUMS = lax.GatherDimensionNumbers(
    offset_dims=(), collapsed_slice_dims=(0,), start_index_map=(0,))


def _permute(x, idx):
  return lax.gather(x, idx[:, None], _GATHER_DNUMS, slice_sizes=(1,),
                    mode=lax.GatherScatterMode.PROMISE_IN_BOUNDS)


def _topk_body(x_hbm, p_hbm, i_hbm, x_v, p_v, i_v):
  wid = lax.axis_index("s") * NC + lax.axis_index("c")
  base = wid * TPW
  pltpu.sync_copy(x_hbm.at[pl.ds(base * N_EXPERTS, TPW * N_EXPERTS)], x_v)

  lane = lax.iota(jnp.int32, L)
  sel8 = lane < TOPK
  rev8 = jnp.where(sel8, (TOPK - 1) - lane, 0)
  shl8 = jnp.where(sel8, 0, lane - TOPK)

  def merge8(ka, va, kb, vb):
    kr = _permute(kb, rev8)
    vr = _permute(vb, rev8)
    take_a = ka >= kr
    return jnp.where(take_a, ka, kr), jnp.where(take_a, va, vr)

  def token_topk(off):
    ks, vs = [], []
    for c in range(N_EXPERTS // L):
      x = x_v[pl.ds(off + c * L, L)]
      sk, sv = plsc.sort_key_val(x, lane + c * L, descending=True)
      ks.append(sk)
      vs.append(sv)
    k01, v01 = merge8(ks[0], vs[0], ks[1], vs[1])
    k23, v23 = merge8(ks[2], vs[2], ks[3], vs[3])
    ck = jnp.where(sel8, k01, _permute(k23, shl8))
    cv = jnp.where(sel8, v01, _permute(v23, shl8))
    fk, fv = plsc.sort_key_val(ck, cv, descending=True)
    m = jnp.max(fk)
    e = jnp.where(sel8, jnp.exp(fk - m), 0.0)
    return e / jnp.sum(e), fv

  def pair_body(t2, carry):
    off = t2 * (2 * N_EXPERTS)
    p_a, v_a = token_topk(off)
    p_b, v_b = token_topk(off + N_EXPERTS)
    pp = jnp.where(sel8, p_a, _permute(p_b, shl8))
    vv = jnp.where(sel8, v_a, _permute(v_b, shl8))
    p_v[pl.ds(t2 * L, L)] = pp
    i_v[pl.ds(t2 * L, L)] = vv
    return carry

  lax.fori_loop(0, TPW // 2, pair_body, 0)

  pltpu.sync_copy(p_v, p_hbm.at[pl.ds(base * TOPK, TPW * TOPK)])
  pltpu.sync_copy(i_v, i_hbm.at[pl.ds(base * TOPK, TPW * TOPK)])


_topk_call = pl.kernel(
    _topk_body,
    out_type=(
        jax.ShapeDtypeStruct((N_TOKENS * TOPK,), jnp.float32),
        jax.ShapeDtypeStruct((N_TOKENS * TOPK,), jnp.int32),
    ),
    mesh=plsc.VectorSubcoreMesh(
        core_axis_name="c", subcore_axis_name="s",
        num_cores=NC, num_subcores=NS),
    scratch_types=[
        pltpu.VMEM((TPW * N_EXPERTS,), jnp.float32),
        pltpu.VMEM((TPW * TOPK,), jnp.float32),
        pltpu.VMEM((TPW * TOPK,), jnp.int32),
    ],
    compiler_params=pltpu.CompilerParams(needs_layout_passes=False),
)


def kernel(gating_logits):
  n, e = gating_logits.shape
  assert n == N_TOKENS and e == N_EXPERTS
  probs_flat, idx_flat = _topk_call(gating_logits.reshape(-1))
  return probs_flat.reshape(n, TOPK), idx_flat.reshape(n, TOPK)
